# baseline (speedup 1.0000x reference)
import jax
import jax.numpy as jnp
from jax import lax
from jax.experimental import pallas as pl
from jax.experimental.pallas import tpu as pltpu
from jax.experimental.pallas import tpu_sc as plsc

N = 10000
E = 320000
D_IN = 128
D_HID = 128
D_OUT = 64

_NC = 2
_NS = 16
_EPT = E // _NS
_CHUNK = 100
_NCHUNK = _EPT // _CHUNK
_NBUF = 4
_PDIST = 2
_DH = D_HID // _NC
_RPT = 624
_TAIL = N - _NS * _RPT
_ZROWS = 104
_DEGW = 16

_mesh = plsc.VectorSubcoreMesh(core_axis_name="c", subcore_axis_name="s")


def _build_sc_agg(with_deg):
    out_type = [jax.ShapeDtypeStruct((_NC, N, _DH), jnp.float32)]
    scratch = [
        pltpu.VMEM((_NCHUNK, _CHUNK), jnp.int32),
        pltpu.VMEM((_NCHUNK, _CHUNK), jnp.int32),
        pltpu.VMEM((_ZROWS, _DH), jnp.float32),
        pltpu.VMEM_SHARED((N, _DH), jnp.float32),
    ] + [pltpu.VMEM((_CHUNK, _DH), jnp.float32) for _ in range(_NBUF)] + [
        pltpu.SemaphoreType.DMA for _ in range(2 * _NBUF)]
    if with_deg:
        out_type.append(jax.ShapeDtypeStruct((_NC, N, _DEGW), jnp.float32))
        scratch += [
            pltpu.VMEM((_CHUNK, _DEGW), jnp.float32),
            pltpu.VMEM((_ZROWS, _DEGW), jnp.float32),
            pltpu.VMEM_SHARED((N, _DEGW), jnp.float32),
        ]

    def body(m_lo, m_hi, src_hbm, dst_hbm, *refs):
        if with_deg:
            (agg_hbm, deg_hbm, src_v, dst_v, zbuf, acc_sh,
             *ring, ones_v, zdeg, deg_sh) = refs
        else:
            (agg_hbm, src_v, dst_v, zbuf, acc_sh, *ring) = refs
        slots = tuple(zip(ring[:_NBUF], ring[_NBUF:2 * _NBUF],
                          ring[2 * _NBUF:3 * _NBUF]))

        cid = lax.axis_index("c")
        sid = lax.axis_index("s")

        pltpu.sync_copy(src_hbm.at[sid], src_v)
        pltpu.sync_copy(dst_hbm.at[sid], dst_v)

        zv = jnp.zeros((16,), jnp.float32)

        @pl.loop(0, _ZROWS)
        def _zfill(i):
            for c in range(_DH // 16):
                zbuf[i, pl.ds(c * 16, 16)] = zv

        base = sid * _RPT
        for t in range(_RPT // _ZROWS):
            pltpu.sync_copy(
                zbuf, acc_sh.at[pl.ds(base + t * _ZROWS, _ZROWS)])

        @pl.when(sid == 0)
        def _ztail():
            pltpu.sync_copy(zbuf.at[pl.ds(0, _TAIL)],
                            acc_sh.at[pl.ds(_NS * _RPT, _TAIL)])

        if with_deg:
            count_cond = jnp.where(cid == 0, sid < _NS // 2, sid >= _NS // 2)

            @pl.loop(0, _ZROWS)
            def _zdfill(i):
                zdeg[i, :] = zv

            for t in range(_RPT // _ZROWS):
                pltpu.sync_copy(
                    zdeg, deg_sh.at[pl.ds(base + t * _ZROWS, _ZROWS)])

            @pl.when(sid == 0)
            def _zdtail():
                pltpu.sync_copy(zdeg.at[pl.ds(0, _TAIL)],
                                deg_sh.at[pl.ds(_NS * _RPT, _TAIL)])

            ov = jnp.full((16,), 1.0, jnp.float32)

            @pl.loop(0, _CHUNK)
            def _ofill(i):
                ones_v[i, :] = ov

        plsc.subcore_barrier()

        def scat_wait(m_ref, jj, b, want_deg):
            rows, _, ssem = slots[b]
            pltpu.make_async_copy(rows, acc_sh.at[dst_v.at[jj]], ssem).wait()
            if want_deg:
                @pl.when(count_cond)
                def _():
                    pltpu.make_async_copy(
                        ones_v, deg_sh.at[dst_v.at[jj]], ssem).wait()

        for m_ref, mine in ((m_lo, 0), (m_hi, 1)):
            @pl.when(cid == mine)
            def _run():
                for k in range(_PDIST):
                    rows, gsem, _ = slots[k]
                    pltpu.async_copy(m_ref.at[src_v.at[k]], rows, gsem)

                @pl.loop(0, _NCHUNK, step=_NBUF)
                def _chunks(g):
                    for b in range(_NBUF):
                        j = g + b
                        c = (b + _PDIST) % _NBUF
                        rows_c, gsem_c, _ = slots[c]

                        @pl.when(j + _PDIST < _NCHUNK)
                        def _prefetch():
                            @pl.when(j + _PDIST >= _NBUF)
                            def _free_slot():
                                scat_wait(m_ref, j + _PDIST - _NBUF, c,
                                          with_deg)
                            pltpu.async_copy(
                                m_ref.at[src_v.at[j + _PDIST]], rows_c,
                                gsem_c)

                        rows_b, gsem_b, ssem_b = slots[b]
                        pltpu.make_async_copy(
                            m_ref.at[src_v.at[j]], rows_b, gsem_b).wait()
                        pltpu.async_copy(rows_b, acc_sh.at[dst_v.at[j]],
                                         ssem_b, add=True)
                        if with_deg:
                            @pl.when(count_cond)
                            def _count():
                                pltpu.async_copy(
                                    ones_v, deg_sh.at[dst_v.at[j]],
                                    ssem_b, add=True)

                for b in range(_NBUF):
                    scat_wait(m_ref, _NCHUNK - _NBUF + b, b, with_deg)

        plsc.subcore_barrier()

        pltpu.sync_copy(acc_sh.at[pl.ds(base, _RPT)],
                        agg_hbm.at[cid, pl.ds(base, _RPT)])

        @pl.when(sid == 0)
        def _wtail():
            pltpu.sync_copy(acc_sh.at[pl.ds(_NS * _RPT, _TAIL)],
                            agg_hbm.at[cid, pl.ds(_NS * _RPT, _TAIL)])

        if with_deg:
            pltpu.sync_copy(deg_sh.at[pl.ds(base, _RPT)],
                            deg_hbm.at[cid, pl.ds(base, _RPT)])

            @pl.when(sid == 0)
            def _wdtail():
                pltpu.sync_copy(deg_sh.at[pl.ds(_NS * _RPT, _TAIL)],
                                deg_hbm.at[cid, pl.ds(_NS * _RPT, _TAIL)])

    return pl.kernel(
        body, out_type=out_type, mesh=_mesh, scratch_types=scratch,
        compiler_params=pltpu.CompilerParams(use_tc_tiling_on_sc=False))


_sc_agg_deg = _build_sc_agg(with_deg=True)
_sc_agg = _build_sc_agg(with_deg=False)


def _mm_body(x_ref, w_ref, lo_ref, hi_ref):
    m = jnp.dot(x_ref[...], w_ref[...], preferred_element_type=jnp.float32)
    lo_ref[...] = m[:, :_DH]
    hi_ref[...] = m[:, _DH:]


_mm = pl.pallas_call(
    _mm_body,
    out_shape=(
        jax.ShapeDtypeStruct((N, _DH), jnp.float32),
        jax.ShapeDtypeStruct((N, _DH), jnp.float32),
    ),
)


def _inv_deg(degp_ref):
    deg = degp_ref[0, :, 0:1] + degp_ref[1, :, 0:1]
    return 1.0 / jnp.maximum(deg, 1.0)


def _layer_body(x_ref, agg_ref, degp_ref, ws_ref, b_ref, wn2_ref,
                h_ref, mlo_ref, mhi_ref):
    neigh = jnp.concatenate([agg_ref[0], agg_ref[1]], axis=1)
    neigh = neigh * _inv_deg(degp_ref)
    h = jnp.dot(x_ref[...], ws_ref[...], preferred_element_type=jnp.float32)
    h = jnp.maximum(h + neigh + b_ref[...][None, :], 0.0)
    h_ref[...] = h
    m = jnp.dot(h, wn2_ref[...], preferred_element_type=jnp.float32)
    mlo_ref[...] = m[:, :_DH]
    mhi_ref[...] = m[:, _DH:]


_layer = pl.pallas_call(
    _layer_body,
    out_shape=(
        jax.ShapeDtypeStruct((N, D_HID), jnp.float32),
        jax.ShapeDtypeStruct((N, _DH), jnp.float32),
        jax.ShapeDtypeStruct((N, _DH), jnp.float32),
    ),
)


def _out_body(h_ref, agg_ref, degp_ref, ws_ref, b_ref, wo_ref, bo_ref,
              o_ref):
    neigh = jnp.concatenate([agg_ref[0], agg_ref[1]], axis=1)
    neigh = neigh * _inv_deg(degp_ref)
    h = jnp.dot(h_ref[...], ws_ref[...], preferred_element_type=jnp.float32)
    h = jnp.maximum(h + neigh + b_ref[...][None, :], 0.0)
    o_ref[...] = (jnp.dot(h, wo_ref[...], preferred_element_type=jnp.float32)
                  + bo_ref[...][None, :])


_out = pl.pallas_call(
    _out_body,
    out_shape=jax.ShapeDtypeStruct((N, D_OUT), jnp.float32),
)


def kernel(x, W_self1, W_neigh1, b1, W_self2, W_neigh2, b2, W_out, b_out,
           edge_index):
    ei = edge_index.astype(jnp.int32)
    src = ei[0].reshape(_NS, _NCHUNK, _CHUNK)
    dst = ei[1].reshape(_NS, _NCHUNK, _CHUNK)

    m1_lo, m1_hi = _mm(x, W_neigh1)
    agg1, degp = _sc_agg_deg(m1_lo, m1_hi, src, dst)
    h1, m2_lo, m2_hi = _layer(x, agg1, degp, W_self1, b1, W_neigh2)
    (agg2,) = _sc_agg(m2_lo, m2_hi, src, dst)
    return _out(h1, agg2, degp, W_self2, b2, W_out, b_out)


# speedup vs baseline: 1.0611x; 1.0611x over previous
import jax
import jax.numpy as jnp
from jax import lax
from jax.experimental import pallas as pl
from jax.experimental.pallas import tpu as pltpu
from jax.experimental.pallas import tpu_sc as plsc

N = 10000
E = 320000
D_IN = 128
D_HID = 128
D_OUT = 64

_NC = 2
_NS = 16
_EPT = E // _NS
_CHUNK = 100
_NCHUNK = _EPT // _CHUNK
_NBUF = 4
_PDIST = 2
_DH = D_HID // _NC
_RPT = 624
_TAIL = N - _NS * _RPT
_ZROWS = 104
_DEGW = 16

_mesh = plsc.VectorSubcoreMesh(core_axis_name="c", subcore_axis_name="s")


def _build_sc_agg(with_deg):
    out_type = [jax.ShapeDtypeStruct((_NC, N, _DH), jnp.float32)]
    scratch = [
        pltpu.VMEM((_NCHUNK, _CHUNK), jnp.int32),
        pltpu.VMEM((_NCHUNK, _CHUNK), jnp.int32),
        pltpu.VMEM((_ZROWS, _DH), jnp.float32),
        pltpu.VMEM_SHARED((N, _DH), jnp.float32),
    ] + [pltpu.VMEM((_CHUNK, _DH), jnp.float32) for _ in range(_NBUF)] + [
        pltpu.SemaphoreType.DMA for _ in range(2 * _NBUF)]
    if with_deg:
        out_type.append(jax.ShapeDtypeStruct((_NC, N, _DEGW), jnp.float32))
        scratch += [
            pltpu.VMEM((_CHUNK, _DEGW), jnp.float32),
            pltpu.VMEM((_ZROWS, _DEGW), jnp.float32),
            pltpu.VMEM_SHARED((N, _DEGW), jnp.float32),
        ]

    def body(m_lo, m_hi, src_hbm, dst_hbm, *refs):
        if with_deg:
            (agg_hbm, deg_hbm, src_v, dst_v, zbuf, acc_sh,
             *ring, ones_v, zdeg, deg_sh) = refs
        else:
            (agg_hbm, src_v, dst_v, zbuf, acc_sh, *ring) = refs
        slots = tuple(zip(ring[:_NBUF], ring[_NBUF:2 * _NBUF],
                          ring[2 * _NBUF:3 * _NBUF]))

        cid = lax.axis_index("c")
        sid = lax.axis_index("s")

        pltpu.sync_copy(src_hbm.at[sid], src_v)
        pltpu.sync_copy(dst_hbm.at[sid], dst_v)

        zv = jnp.zeros((16,), jnp.float32)

        @pl.loop(0, _ZROWS)
        def _zfill(i):
            for c in range(_DH // 16):
                zbuf[i, pl.ds(c * 16, 16)] = zv

        base = sid * _RPT
        for t in range(_RPT // _ZROWS):
            pltpu.sync_copy(
                zbuf, acc_sh.at[pl.ds(base + t * _ZROWS, _ZROWS)])

        @pl.when(sid == 0)
        def _ztail():
            pltpu.sync_copy(zbuf.at[pl.ds(0, _TAIL)],
                            acc_sh.at[pl.ds(_NS * _RPT, _TAIL)])

        if with_deg:
            @pl.loop(0, _ZROWS)
            def _zdfill(i):
                zdeg[i, :] = zv

            for t in range(_RPT // _ZROWS):
                pltpu.sync_copy(
                    zdeg, deg_sh.at[pl.ds(base + t * _ZROWS, _ZROWS)])

            @pl.when(sid == 0)
            def _zdtail():
                pltpu.sync_copy(zdeg.at[pl.ds(0, _TAIL)],
                                deg_sh.at[pl.ds(_NS * _RPT, _TAIL)])

            ov = jnp.full((16,), 1.0, jnp.float32)

            @pl.loop(0, _CHUNK)
            def _ofill(i):
                ones_v[i, :] = ov

        plsc.subcore_barrier()

        for m_ref, mine in ((m_lo, 0), (m_hi, 1)):
            cnt_lo = 0 if mine == 0 else _NCHUNK // 2
            cnt_hi = _NCHUNK // 2 if mine == 0 else _NCHUNK

            @pl.when(cid == mine)
            def _run():
                for k, (rows, gsem, _) in enumerate(slots):
                    pltpu.async_copy(m_ref.at[src_v.at[k]], rows, gsem)

                @pl.loop(0, _NCHUNK, step=_NBUF)
                def _chunks(g):
                    for b, (rows, gsem, _) in enumerate(slots):
                        j = g + b
                        pltpu.make_async_copy(
                            m_ref.at[src_v.at[j]], rows, gsem).wait()
                        pltpu.sync_copy(rows, acc_sh.at[dst_v.at[j]],
                                        add=True)
                        if with_deg:
                            @pl.when((j >= cnt_lo) & (j < cnt_hi))
                            def _count():
                                pltpu.sync_copy(
                                    ones_v, deg_sh.at[dst_v.at[j]],
                                    add=True)

                        @pl.when(j + _NBUF < _NCHUNK)
                        def _():
                            pltpu.async_copy(m_ref.at[src_v.at[j + _NBUF]],
                                             rows, gsem)

        plsc.subcore_barrier()

        pltpu.sync_copy(acc_sh.at[pl.ds(base, _RPT)],
                        agg_hbm.at[cid, pl.ds(base, _RPT)])

        @pl.when(sid == 0)
        def _wtail():
            pltpu.sync_copy(acc_sh.at[pl.ds(_NS * _RPT, _TAIL)],
                            agg_hbm.at[cid, pl.ds(_NS * _RPT, _TAIL)])

        if with_deg:
            pltpu.sync_copy(deg_sh.at[pl.ds(base, _RPT)],
                            deg_hbm.at[cid, pl.ds(base, _RPT)])

            @pl.when(sid == 0)
            def _wdtail():
                pltpu.sync_copy(deg_sh.at[pl.ds(_NS * _RPT, _TAIL)],
                                deg_hbm.at[cid, pl.ds(_NS * _RPT, _TAIL)])

    return pl.kernel(
        body, out_type=out_type, mesh=_mesh, scratch_types=scratch,
        compiler_params=pltpu.CompilerParams(use_tc_tiling_on_sc=False))


_sc_agg_deg = _build_sc_agg(with_deg=True)
_sc_agg = _build_sc_agg(with_deg=False)


def _mm_body(x_ref, w_ref, lo_ref, hi_ref):
    m = jnp.dot(x_ref[...], w_ref[...], preferred_element_type=jnp.float32)
    lo_ref[...] = m[:, :_DH]
    hi_ref[...] = m[:, _DH:]


_mm = pl.pallas_call(
    _mm_body,
    out_shape=(
        jax.ShapeDtypeStruct((N, _DH), jnp.float32),
        jax.ShapeDtypeStruct((N, _DH), jnp.float32),
    ),
)


def _inv_deg(degp_ref):
    deg = degp_ref[0, :, 0:1] + degp_ref[1, :, 0:1]
    return 1.0 / jnp.maximum(deg, 1.0)


def _layer_body(x_ref, agg_ref, degp_ref, ws_ref, b_ref, wn2_ref,
                h_ref, mlo_ref, mhi_ref):
    neigh = jnp.concatenate([agg_ref[0], agg_ref[1]], axis=1)
    neigh = neigh * _inv_deg(degp_ref)
    h = jnp.dot(x_ref[...], ws_ref[...], preferred_element_type=jnp.float32)
    h = jnp.maximum(h + neigh + b_ref[...][None, :], 0.0)
    h_ref[...] = h
    m = jnp.dot(h, wn2_ref[...], preferred_element_type=jnp.float32)
    mlo_ref[...] = m[:, :_DH]
    mhi_ref[...] = m[:, _DH:]


_layer = pl.pallas_call(
    _layer_body,
    out_shape=(
        jax.ShapeDtypeStruct((N, D_HID), jnp.float32),
        jax.ShapeDtypeStruct((N, _DH), jnp.float32),
        jax.ShapeDtypeStruct((N, _DH), jnp.float32),
    ),
)


def _out_body(h_ref, agg_ref, degp_ref, ws_ref, b_ref, wo_ref, bo_ref,
              o_ref):
    neigh = jnp.concatenate([agg_ref[0], agg_ref[1]], axis=1)
    neigh = neigh * _inv_deg(degp_ref)
    h = jnp.dot(h_ref[...], ws_ref[...], preferred_element_type=jnp.float32)
    h = jnp.maximum(h + neigh + b_ref[...][None, :], 0.0)
    o_ref[...] = (jnp.dot(h, wo_ref[...], preferred_element_type=jnp.float32)
                  + bo_ref[...][None, :])


_out = pl.pallas_call(
    _out_body,
    out_shape=jax.ShapeDtypeStruct((N, D_OUT), jnp.float32),
)


def kernel(x, W_self1, W_neigh1, b1, W_self2, W_neigh2, b2, W_out, b_out,
           edge_index):
    ei = edge_index.astype(jnp.int32)
    src = ei[0].reshape(_NS, _NCHUNK, _CHUNK)
    dst = ei[1].reshape(_NS, _NCHUNK, _CHUNK)

    m1_lo, m1_hi = _mm(x, W_neigh1)
    agg1, degp = _sc_agg_deg(m1_lo, m1_hi, src, dst)
    h1, m2_lo, m2_hi = _layer(x, agg1, degp, W_self1, b1, W_neigh2)
    (agg2,) = _sc_agg(m2_lo, m2_hi, src, dst)
    return _out(h1, agg2, degp, W_self2, b2, W_out, b_out)
